# Initial kernel scaffold; baseline (speedup 1.0000x reference)
import jax
import jax.numpy as jnp
from jax.experimental import pallas as pl


def kernel(subtoken_ids, segment_ids, word_labels):
    raise NotImplementedError("write your pallas kernel here")


# speedup vs baseline: 9.8529x; 9.8529x over previous
import functools

import jax
import jax.numpy as jnp
from jax import lax
from jax.experimental import pallas as pl
from jax.experimental.pallas import tpu as pltpu
from jax.experimental.pallas import tpu_sc as plsc

_SEQ = 128
_CLS = 101
_SEP = 102
_N_WORDS = 16384
_N_TOK = 32768
_N_LABELS = _N_TOK + 2

_CHUNK = 1024
_WIN = _CHUNK + 16
_NW = 32


def _body(st_hbm, seg_hbm, wl_hbm, ids_hbm, mask_hbm, type_hbm, lab_hbm,
          table_v, win_v, out_v):
    c = lax.axis_index("c")
    s = lax.axis_index("s")
    wid = s * 2 + c
    base = wid * _CHUNK

    pltpu.sync_copy(wl_hbm, table_v)
    win0 = pl.multiple_of(jnp.maximum(base - 16, 0), 16)
    pltpu.sync_copy(seg_hbm.at[pl.ds(win0, _WIN)], win_v)

    iota = lax.iota(jnp.int32, 16)
    zero = jnp.zeros((16,), jnp.int32)

    def group(i, carry):
        p = base + i * 16 + iota
        loc = jnp.clip(p - 1 - win0, 0, _WIN - 1)
        segv = plsc.load_gather(win_v, [loc])
        vals = plsc.load_gather(table_v, [segv]) + 1
        vals = jnp.where(p == 0, zero, vals)
        vals = jnp.where(p >= _N_LABELS - 1, zero, vals)
        out_v[pl.ds(i * 16, 16)] = vals
        return carry

    lax.fori_loop(0, _WIN // 16, group, 0)

    pltpu.sync_copy(out_v.at[pl.ds(0, _CHUNK)], lab_hbm.at[pl.ds(base, _CHUNK)])

    @pl.when(wid == _NW - 1)
    def _tail():
        pltpu.sync_copy(out_v.at[pl.ds(_CHUNK, 2)],
                        lab_hbm.at[pl.ds(_NW * _CHUNK, 2)])

    @pl.when(wid == 1)
    def _ids():
        pltpu.sync_copy(st_hbm.at[pl.ds(0, _SEQ)], win_v.at[pl.ds(0, _SEQ)])
        for i in range(_SEQ // 16):
            p = i * 16 + iota
            loc = jnp.clip(p - 1, 0, _SEQ - 1)
            v = plsc.load_gather(win_v, [loc])
            v = jnp.where(p == 0, jnp.full((16,), _CLS, jnp.int32), v)
            v = jnp.where(p == _SEQ - 1, jnp.full((16,), _SEP, jnp.int32), v)
            out_v[pl.ds(i * 16, 16)] = v
        pltpu.sync_copy(out_v.at[pl.ds(0, _SEQ)], ids_hbm)

    @pl.when(wid == 2)
    def _mask():
        one = jnp.ones((16,), jnp.int32)
        for i in range(_SEQ // 16):
            out_v[pl.ds(i * 16, 16)] = one
        pltpu.sync_copy(out_v.at[pl.ds(0, _SEQ)], mask_hbm)

    @pl.when(wid == 3)
    def _type():
        for i in range(_SEQ // 16):
            out_v[pl.ds(i * 16, 16)] = zero
        pltpu.sync_copy(out_v.at[pl.ds(0, _SEQ)], type_hbm)


@jax.jit
def _run(subtoken_ids, seg32, wl32):
    i32 = jnp.int32
    k = functools.partial(
        pl.kernel,
        out_type=(
            jax.ShapeDtypeStruct((_SEQ,), i32),
            jax.ShapeDtypeStruct((_SEQ,), i32),
            jax.ShapeDtypeStruct((_SEQ,), i32),
            jax.ShapeDtypeStruct((_N_LABELS,), i32),
        ),
        mesh=plsc.VectorSubcoreMesh(core_axis_name="c", subcore_axis_name="s"),
        compiler_params=pltpu.CompilerParams(needs_layout_passes=False),
        scratch_types=[
            pltpu.VMEM((_N_WORDS,), i32),
            pltpu.VMEM((_WIN,), i32),
            pltpu.VMEM((_WIN,), i32),
        ],
    )(_body)
    return k(subtoken_ids, seg32, wl32)


def kernel(subtoken_ids, segment_ids, word_labels):
    seg32 = segment_ids.astype(jnp.int32)
    wl32 = word_labels.astype(jnp.int32)
    return _run(subtoken_ids, seg32, wl32)


# speedup vs baseline: 11.1345x; 1.1301x over previous
import functools

import jax
import jax.numpy as jnp
from jax import lax
from jax.experimental import pallas as pl
from jax.experimental.pallas import tpu as pltpu
from jax.experimental.pallas import tpu_sc as plsc

_SEQ = 128
_CLS = 101
_SEP = 102
_N_WORDS = 16384
_N_TOK = 32768
_N_LABELS = _N_TOK + 2

_CHUNK = 1024
_WIN = _CHUNK + 16
_NW = 32
_SLOT = 1024


def _body(st_hbm, seg_hbm, wl_hbm, ids_hbm, mask_hbm, type_hbm, lab_hbm,
          table_v, win_v, out_v):
    c = lax.axis_index("c")
    s = lax.axis_index("s")
    wid = s * 2 + c
    base = wid * _CHUNK

    win0 = pl.multiple_of(jnp.maximum(base - 16, 0), 16)
    pltpu.sync_copy(seg_hbm.at[pl.ds(win0, _WIN)], win_v)

    lo = jnp.min(win_v[pl.ds(0, 16)])
    hi = jnp.max(win_v[pl.ds(_WIN - 16, 16)])
    slot0 = lo // _SLOT
    nslots = hi // _SLOT - slot0 + 1
    slot_base = slot0 * _SLOT

    def copy_slot(k, carry):
        src = pl.multiple_of((slot0 + k) * _SLOT, _SLOT)
        pltpu.sync_copy(wl_hbm.at[pl.ds(src, _SLOT)],
                        table_v.at[pl.ds(k * _SLOT, _SLOT)])
        return carry

    lax.fori_loop(0, nslots, copy_slot, 0)

    iota = lax.iota(jnp.int32, 16)
    zero = jnp.zeros((16,), jnp.int32)

    @plsc.parallel_loop(0, _WIN // 16, unroll=5)
    def group(i):
        p = base + i * 16 + iota
        loc = jnp.clip(p - 1 - win0, 0, _WIN - 1)
        segv = plsc.load_gather(win_v, [loc])
        vals = plsc.load_gather(table_v, [segv - slot_base]) + 1
        vals = jnp.where(p == 0, zero, vals)
        vals = jnp.where(p >= _N_LABELS - 1, zero, vals)
        out_v[pl.ds(i * 16, 16)] = vals

    pltpu.sync_copy(out_v.at[pl.ds(0, _CHUNK)], lab_hbm.at[pl.ds(base, _CHUNK)])

    @pl.when(wid == _NW - 1)
    def _tail():
        pltpu.sync_copy(out_v.at[pl.ds(_CHUNK, 2)],
                        lab_hbm.at[pl.ds(_NW * _CHUNK, 2)])

    @pl.when(wid == 1)
    def _ids():
        pltpu.sync_copy(st_hbm.at[pl.ds(0, _SEQ)], win_v.at[pl.ds(0, _SEQ)])
        for i in range(_SEQ // 16):
            p = i * 16 + iota
            loc = jnp.clip(p - 1, 0, _SEQ - 1)
            v = plsc.load_gather(win_v, [loc])
            v = jnp.where(p == 0, jnp.full((16,), _CLS, jnp.int32), v)
            v = jnp.where(p == _SEQ - 1, jnp.full((16,), _SEP, jnp.int32), v)
            out_v[pl.ds(i * 16, 16)] = v
        pltpu.sync_copy(out_v.at[pl.ds(0, _SEQ)], ids_hbm)

    @pl.when(wid == 2)
    def _mask():
        one = jnp.ones((16,), jnp.int32)
        for i in range(_SEQ // 16):
            out_v[pl.ds(i * 16, 16)] = one
        pltpu.sync_copy(out_v.at[pl.ds(0, _SEQ)], mask_hbm)

    @pl.when(wid == 3)
    def _type():
        for i in range(_SEQ // 16):
            out_v[pl.ds(i * 16, 16)] = zero
        pltpu.sync_copy(out_v.at[pl.ds(0, _SEQ)], type_hbm)


@jax.jit
def _run(subtoken_ids, seg32, wl32):
    i32 = jnp.int32
    k = functools.partial(
        pl.kernel,
        out_type=(
            jax.ShapeDtypeStruct((_SEQ,), i32),
            jax.ShapeDtypeStruct((_SEQ,), i32),
            jax.ShapeDtypeStruct((_SEQ,), i32),
            jax.ShapeDtypeStruct((_N_LABELS,), i32),
        ),
        mesh=plsc.VectorSubcoreMesh(core_axis_name="c", subcore_axis_name="s"),
        compiler_params=pltpu.CompilerParams(needs_layout_passes=False),
        scratch_types=[
            pltpu.VMEM((_N_WORDS,), i32),
            pltpu.VMEM((_WIN,), i32),
            pltpu.VMEM((_WIN,), i32),
        ],
    )(_body)
    return k(subtoken_ids, seg32, wl32)


def kernel(subtoken_ids, segment_ids, word_labels):
    seg32 = segment_ids.astype(jnp.int32)
    wl32 = word_labels.astype(jnp.int32)
    return _run(subtoken_ids, seg32, wl32)


# speedup vs baseline: 11.1767x; 1.0038x over previous
import functools

import jax
import jax.numpy as jnp
from jax import lax
from jax.experimental import pallas as pl
from jax.experimental.pallas import tpu as pltpu
from jax.experimental.pallas import tpu_sc as plsc

_SEQ = 128
_CLS = 101
_SEP = 102
_N_WORDS = 16384
_N_TOK = 32768
_N_LABELS = _N_TOK + 2

_CHUNK = 1024
_WIN = _CHUNK + 16
_NW = 32
_SLOT = 1024


def _body(st_hbm, seg_hbm, wl_hbm, ids_hbm, mask_hbm, type_hbm, lab_hbm,
          table_v, win_v, out_v):
    c = lax.axis_index("c")
    s = lax.axis_index("s")
    wid = s * 2 + c
    base = wid * _CHUNK

    win0 = pl.multiple_of(jnp.maximum(base - 16, 0), 16)
    pltpu.sync_copy(seg_hbm.at[pl.ds(win0, _WIN)], win_v)

    lo = jnp.min(win_v[pl.ds(0, 16)])
    hi = jnp.max(win_v[pl.ds(_WIN - 16, 16)])
    slot0 = lo // _SLOT
    nslots = hi // _SLOT - slot0 + 1
    slot_base = slot0 * _SLOT

    def copy_slot(k, carry):
        src = pl.multiple_of((slot0 + k) * _SLOT, _SLOT)
        pltpu.sync_copy(wl_hbm.at[pl.ds(src, _SLOT)],
                        table_v.at[pl.ds(k * _SLOT, _SLOT)])
        return carry

    lax.fori_loop(0, nslots, copy_slot, 0)

    iota = lax.iota(jnp.int32, 16)
    zero = jnp.zeros((16,), jnp.int32)

    @plsc.parallel_loop(0, _WIN // 16, unroll=5)
    def group(i):
        p = base + i * 16 + iota
        loc = jnp.clip(p - 1 - win0, 0, _WIN - 1)
        segv = plsc.load_gather(win_v, [loc])
        vals = plsc.load_gather(table_v, [segv - slot_base]) + 1
        vals = jnp.where(p == 0, zero, vals)
        vals = jnp.where(p >= _N_LABELS - 1, zero, vals)
        out_v[pl.ds(i * 16, 16)] = vals

    pltpu.sync_copy(out_v.at[pl.ds(0, _CHUNK)], lab_hbm.at[pl.ds(base, _CHUNK)])

    @pl.when(wid == _NW - 1)
    def _tail():
        pltpu.sync_copy(out_v.at[pl.ds(_CHUNK, 2)],
                        lab_hbm.at[pl.ds(_NW * _CHUNK, 2)])

    @pl.when(wid == 1)
    def _ids():
        pltpu.sync_copy(st_hbm.at[pl.ds(0, _SEQ)], win_v.at[pl.ds(0, _SEQ)])

        def idgrp(i, carry):
            p = i * 16 + iota
            loc = jnp.clip(p - 1, 0, _SEQ - 1)
            v = plsc.load_gather(win_v, [loc])
            v = jnp.where(p == 0, jnp.full((16,), _CLS, jnp.int32), v)
            v = jnp.where(p == _SEQ - 1, jnp.full((16,), _SEP, jnp.int32), v)
            out_v[pl.ds(i * 16, 16)] = v
            return carry

        lax.fori_loop(0, _SEQ // 16, idgrp, 0)
        pltpu.sync_copy(out_v.at[pl.ds(0, _SEQ)], ids_hbm)

    @pl.when(wid == 2)
    def _mask():
        one = jnp.ones((16,), jnp.int32)

        def mgrp(i, carry):
            out_v[pl.ds(i * 16, 16)] = one
            return carry

        lax.fori_loop(0, _SEQ // 16, mgrp, 0)
        pltpu.sync_copy(out_v.at[pl.ds(0, _SEQ)], mask_hbm)

    @pl.when(wid == 3)
    def _type():
        def zgrp(i, carry):
            out_v[pl.ds(i * 16, 16)] = zero
            return carry

        lax.fori_loop(0, _SEQ // 16, zgrp, 0)
        pltpu.sync_copy(out_v.at[pl.ds(0, _SEQ)], type_hbm)


@jax.jit
def _run(subtoken_ids, seg32, wl32):
    i32 = jnp.int32
    k = functools.partial(
        pl.kernel,
        out_type=(
            jax.ShapeDtypeStruct((_SEQ,), i32),
            jax.ShapeDtypeStruct((_SEQ,), i32),
            jax.ShapeDtypeStruct((_SEQ,), i32),
            jax.ShapeDtypeStruct((_N_LABELS,), i32),
        ),
        mesh=plsc.VectorSubcoreMesh(core_axis_name="c", subcore_axis_name="s"),
        compiler_params=pltpu.CompilerParams(needs_layout_passes=False,
                                             skip_device_barrier=True),
        scratch_types=[
            pltpu.VMEM((_N_WORDS,), i32),
            pltpu.VMEM((_WIN,), i32),
            pltpu.VMEM((_WIN,), i32),
        ],
    )(_body)
    return k(subtoken_ids, seg32, wl32)


def kernel(subtoken_ids, segment_ids, word_labels):
    seg32 = segment_ids.astype(jnp.int32)
    wl32 = word_labels.astype(jnp.int32)
    return _run(subtoken_ids, seg32, wl32)


# speedup vs baseline: 11.3885x; 1.0189x over previous
import functools

import jax
import jax.numpy as jnp
from jax import lax
from jax.experimental import pallas as pl
from jax.experimental.pallas import tpu as pltpu
from jax.experimental.pallas import tpu_sc as plsc

_SEQ = 128
_CLS = 101
_SEP = 102
_N_WORDS = 16384
_N_TOK = 32768
_N_LABELS = _N_TOK + 2

_CHUNK = 1024
_WIN = _CHUNK + 16
_NW = 32
_SLOT = 1024


def _body(st_hbm, seg_hbm, wl_hbm, ids_hbm, mask_hbm, type_hbm, lab_hbm,
          table_v, win_v, idx_v, out_v, tsem, osem):
    c = lax.axis_index("c")
    s = lax.axis_index("s")
    wid = s * 2 + c
    base = wid * _CHUNK

    win0 = pl.multiple_of(jnp.maximum(base - 16, 0), 16)
    pltpu.sync_copy(seg_hbm.at[pl.ds(win0, _WIN)], win_v)

    lo = jnp.min(win_v[pl.ds(0, 16)])
    hi = jnp.max(win_v[pl.ds(_WIN - 16, 16)])
    slot0 = lo // _SLOT
    nslots = hi // _SLOT - slot0 + 1
    slot_base = slot0 * _SLOT

    def fire_slot(k, carry):
        src = pl.multiple_of((slot0 + k) * _SLOT, _SLOT)
        pltpu.async_copy(wl_hbm.at[pl.ds(src, _SLOT)],
                         table_v.at[pl.ds(k * _SLOT, _SLOT)], tsem)
        return carry

    lax.fori_loop(0, nslots, fire_slot, 0)

    iota = lax.iota(jnp.int32, 16)
    zero = jnp.zeros((16,), jnp.int32)

    @plsc.parallel_loop(0, _WIN // 16, unroll=5)
    def build(i):
        p = base + i * 16 + iota
        loc = jnp.clip(p - 1 - win0, 0, _WIN - 1)
        idx_v[pl.ds(i * 16, 16)] = plsc.load_gather(win_v, [loc])

    def drain_slot(k, carry):
        src = pl.multiple_of((slot0 + k) * _SLOT, _SLOT)
        pltpu.make_async_copy(wl_hbm.at[pl.ds(src, _SLOT)],
                              table_v.at[pl.ds(k * _SLOT, _SLOT)], tsem).wait()
        return carry

    lax.fori_loop(0, nslots, drain_slot, 0)

    half = _CHUNK // 2

    @plsc.parallel_loop(0, 32, unroll=4)
    def gather_a(i):
        p = base + i * 16 + iota
        segv = idx_v[pl.ds(i * 16, 16)]
        vals = plsc.load_gather(table_v, [segv - slot_base]) + 1
        out_v[pl.ds(i * 16, 16)] = jnp.where(p == 0, zero, vals)

    cp_a = pltpu.async_copy(out_v.at[pl.ds(0, half)],
                            lab_hbm.at[pl.ds(base, half)], osem)

    @plsc.parallel_loop(32, _WIN // 16, unroll=3)
    def gather_b(i):
        p = base + i * 16 + iota
        segv = idx_v[pl.ds(i * 16, 16)]
        vals = plsc.load_gather(table_v, [segv - slot_base]) + 1
        out_v[pl.ds(i * 16, 16)] = jnp.where(p >= _N_LABELS - 1, zero, vals)

    cp_b = pltpu.async_copy(out_v.at[pl.ds(half, half)],
                            lab_hbm.at[pl.ds(base + half, half)], osem)
    cp_a.wait()
    cp_b.wait()

    @pl.when(wid == _NW - 1)
    def _tail():
        pltpu.sync_copy(out_v.at[pl.ds(_CHUNK, 2)],
                        lab_hbm.at[pl.ds(_NW * _CHUNK, 2)])

    @pl.when(wid == 1)
    def _ids():
        pltpu.sync_copy(st_hbm.at[pl.ds(0, _SEQ)], win_v.at[pl.ds(0, _SEQ)])

        def idgrp(i, carry):
            p = i * 16 + iota
            loc = jnp.clip(p - 1, 0, _SEQ - 1)
            v = plsc.load_gather(win_v, [loc])
            v = jnp.where(p == 0, jnp.full((16,), _CLS, jnp.int32), v)
            v = jnp.where(p == _SEQ - 1, jnp.full((16,), _SEP, jnp.int32), v)
            out_v[pl.ds(i * 16, 16)] = v
            return carry

        lax.fori_loop(0, _SEQ // 16, idgrp, 0)
        pltpu.sync_copy(out_v.at[pl.ds(0, _SEQ)], ids_hbm)

    @pl.when(wid == 2)
    def _mask():
        one = jnp.ones((16,), jnp.int32)

        def mgrp(i, carry):
            out_v[pl.ds(i * 16, 16)] = one
            return carry

        lax.fori_loop(0, _SEQ // 16, mgrp, 0)
        pltpu.sync_copy(out_v.at[pl.ds(0, _SEQ)], mask_hbm)

    @pl.when(wid == 3)
    def _type():
        def zgrp(i, carry):
            out_v[pl.ds(i * 16, 16)] = zero
            return carry

        lax.fori_loop(0, _SEQ // 16, zgrp, 0)
        pltpu.sync_copy(out_v.at[pl.ds(0, _SEQ)], type_hbm)


@jax.jit
def _run(subtoken_ids, seg32, wl32):
    i32 = jnp.int32
    k = functools.partial(
        pl.kernel,
        out_type=(
            jax.ShapeDtypeStruct((_SEQ,), i32),
            jax.ShapeDtypeStruct((_SEQ,), i32),
            jax.ShapeDtypeStruct((_SEQ,), i32),
            jax.ShapeDtypeStruct((_N_LABELS,), i32),
        ),
        mesh=plsc.VectorSubcoreMesh(core_axis_name="c", subcore_axis_name="s"),
        compiler_params=pltpu.CompilerParams(needs_layout_passes=False,
                                             skip_device_barrier=True),
        scratch_types=[
            pltpu.VMEM((_N_WORDS,), i32),
            pltpu.VMEM((_WIN,), i32),
            pltpu.VMEM((_WIN,), i32),
            pltpu.VMEM((_WIN,), i32),
            pltpu.SemaphoreType.DMA,
            pltpu.SemaphoreType.DMA,
        ],
    )(_body)
    return k(subtoken_ids, seg32, wl32)


def kernel(subtoken_ids, segment_ids, word_labels):
    seg32 = segment_ids.astype(jnp.int32)
    wl32 = word_labels.astype(jnp.int32)
    return _run(subtoken_ids, seg32, wl32)
